# Initial kernel scaffold; baseline (speedup 1.0000x reference)
import jax
import jax.numpy as jnp
from jax.experimental import pallas as pl


def kernel(x, edge_index, batch, base, cond, W1, b1, W2, b2, W3, b3, W4, b4, W5, b5, Wl1, bl1, Wl2, bl2):
    raise NotImplementedError("write your pallas kernel here")


# speedup vs baseline: 15.3428x; 15.3428x over previous
import functools

import jax
import jax.numpy as jnp
from jax import lax
from jax.experimental import pallas as pl
from jax.experimental.pallas import tpu as pltpu
from jax.experimental.pallas import tpu_sc as plsc

N = 10000
E = 320000
D = 128
B = 8
L = 16
D_COND = 16
N_C = 8
NPG = N // B

N_PAD = 10240
NW = 32
CHUNK = 128
NCH = (E + NW * CHUNK - 1) // (NW * CHUNK)
E_PAD = NW * NCH * CHUNK
ROWS_PER_TILE = N_PAD // 16
BLK = 1024


@functools.cache
def _sc_mesh():
    return plsc.VectorSubcoreMesh(core_axis_name="c", subcore_axis_name="s",
                                  num_cores=2, num_subcores=16)


@functools.cache
def _sc_degree_fn():
    @functools.partial(
        pl.kernel,
        out_type=jax.ShapeDtypeStruct((2, N_PAD, D), jnp.float32),
        mesh=_sc_mesh(),
        scratch_types=[
            pltpu.VMEM((NCH, CHUNK), jnp.int32),
            pltpu.VMEM((CHUNK, D), jnp.float32),
            pltpu.VMEM_SHARED((N_PAD, D), jnp.float32),
        ],
    )
    def _sc_degree(dst_hbm, e1_hbm, zeros_hbm, out_hbm, dst_v, e1_v, deg_sh):
        c = lax.axis_index("c")
        s = lax.axis_index("s")
        wid = c * 16 + s
        pltpu.sync_copy(dst_hbm.at[wid], dst_v)
        pltpu.sync_copy(e1_hbm, e1_v)
        pltpu.sync_copy(zeros_hbm, deg_sh.at[pl.ds(s * ROWS_PER_TILE, ROWS_PER_TILE)])
        plsc.subcore_barrier()

        def body(j, carry):
            pltpu.sync_copy(e1_v, deg_sh.at[dst_v.at[j]], add=True)
            return carry

        lax.fori_loop(0, NCH, body, 0)
        plsc.subcore_barrier()
        sl = pl.ds(s * ROWS_PER_TILE, ROWS_PER_TILE)
        pltpu.sync_copy(deg_sh.at[sl], out_hbm.at[c].at[sl])

    return _sc_degree


@functools.cache
def _sc_agg_fn():
    @functools.partial(
        pl.kernel,
        out_type=jax.ShapeDtypeStruct((2, N_PAD, D), jnp.float32),
        mesh=_sc_mesh(),
        scratch_types=[
            pltpu.VMEM((NCH, CHUNK), jnp.int32),
            pltpu.VMEM((NCH, CHUNK), jnp.int32),
            pltpu.VMEM((CHUNK, D), jnp.float32),
            pltpu.VMEM_SHARED((N_PAD, D), jnp.float32),
        ],
    )
    def _sc_agg(g_hbm, src_hbm, dst_hbm, zeros_hbm, out_hbm,
                src_v, dst_v, rows_v, acc_sh):
        c = lax.axis_index("c")
        s = lax.axis_index("s")
        wid = c * 16 + s
        pltpu.sync_copy(src_hbm.at[wid], src_v)
        pltpu.sync_copy(dst_hbm.at[wid], dst_v)
        pltpu.sync_copy(zeros_hbm, acc_sh.at[pl.ds(s * ROWS_PER_TILE, ROWS_PER_TILE)])
        plsc.subcore_barrier()

        def body(j, carry):
            pltpu.sync_copy(g_hbm.at[src_v.at[j]], rows_v)
            pltpu.sync_copy(rows_v, acc_sh.at[dst_v.at[j]], add=True)
            return carry

        lax.fori_loop(0, NCH, body, 0)
        plsc.subcore_barrier()
        sl = pl.ds(s * ROWS_PER_TILE, ROWS_PER_TILE)
        pltpu.sync_copy(acc_sh.at[sl], out_hbm.at[c].at[sl])

    return _sc_agg


def _tc_first(x_p, W1, deg_parts):
    def body(x_ref, w_ref, deg_ref, g_ref, dinv_ref):
        d = deg_ref[0, :, :1] + deg_ref[1, :, :1]
        dinv = lax.rsqrt(d + 1.0)
        t = jnp.dot(x_ref[...], w_ref[...], preferred_element_type=jnp.float32)
        g_ref[...] = t * dinv
        dinv_ref[...] = dinv

    grid = N_PAD // BLK
    return pl.pallas_call(
        body,
        grid=(grid,),
        in_specs=[
            pl.BlockSpec((BLK, D), lambda i: (i, 0)),
            pl.BlockSpec((D, D), lambda i: (0, 0)),
            pl.BlockSpec((2, BLK, D), lambda i: (0, i, 0)),
        ],
        out_specs=[
            pl.BlockSpec((BLK, D), lambda i: (i, 0)),
            pl.BlockSpec((BLK, 1), lambda i: (i, 0)),
        ],
        out_shape=[
            jax.ShapeDtypeStruct((N_PAD, D), jnp.float32),
            jax.ShapeDtypeStruct((N_PAD, 1), jnp.float32),
        ],
    )(x_p, W1, deg_parts)


def _tc_mid(acc, g_in, dinv, b_prev, W_next):
    def body(acc_ref, g_ref, dinv_ref, b_ref, w_ref, o_ref):
        h = (acc_ref[0] + acc_ref[1] + g_ref[...]) * dinv_ref[...] + b_ref[...]
        a = jnp.maximum(h, 0.0)
        t = jnp.dot(a, w_ref[...], preferred_element_type=jnp.float32)
        o_ref[...] = t * dinv_ref[...]

    grid = N_PAD // BLK
    return pl.pallas_call(
        body,
        grid=(grid,),
        in_specs=[
            pl.BlockSpec((2, BLK, D), lambda i: (0, i, 0)),
            pl.BlockSpec((BLK, D), lambda i: (i, 0)),
            pl.BlockSpec((BLK, 1), lambda i: (i, 0)),
            pl.BlockSpec((1, D), lambda i: (0, 0)),
            pl.BlockSpec((D, D), lambda i: (0, 0)),
        ],
        out_specs=pl.BlockSpec((BLK, D), lambda i: (i, 0)),
        out_shape=jax.ShapeDtypeStruct((N_PAD, D), jnp.float32),
    )(acc, g_in, dinv, b_prev.reshape(1, D), W_next)


def _tc_last(acc, g_in, dinv, b5):
    def body(acc_ref, g_ref, dinv_ref, b_ref, o_ref):
        o_ref[...] = (acc_ref[0] + acc_ref[1] + g_ref[...]) * dinv_ref[...] + b_ref[...]

    grid = N_PAD // BLK
    return pl.pallas_call(
        body,
        grid=(grid,),
        in_specs=[
            pl.BlockSpec((2, BLK, D), lambda i: (0, i, 0)),
            pl.BlockSpec((BLK, D), lambda i: (i, 0)),
            pl.BlockSpec((BLK, 1), lambda i: (i, 0)),
            pl.BlockSpec((1, D), lambda i: (0, 0)),
        ],
        out_specs=pl.BlockSpec((BLK, D), lambda i: (i, 0)),
        out_shape=jax.ShapeDtypeStruct((N_PAD, D), jnp.float32),
    )(acc, g_in, dinv, b5.reshape(1, D))


def _tc_head(h5, batch_p, base, cond, Wl1, bl1, Wl2, bl2):
    def body(base_ref, h_ref, batch_ref, cond_ref, wl1_ref, bl1_ref,
             wl2_ref, bl2_ref, o_ref, z_ref, z2_ref):
        row_id = lax.broadcasted_iota(jnp.int32, (B, N_PAD), 0)
        sel = jnp.where(batch_ref[...] == row_id, 1.0, 0.0)
        cnts = jnp.sum(sel, axis=1, keepdims=True)
        xg = jnp.dot(sel, h_ref[...], preferred_element_type=jnp.float32) / cnts
        z_ref[:, L * D:] = xg
        for b in range(B):
            for l in range(L):
                idx = base_ref[b, l]
                row = h_ref[pl.ds(b * NPG + idx, 1), :]
                m = jnp.where(idx != 0, 1.0, 0.0)
                for j in range(l):
                    m = m * jnp.where(base_ref[b, j] == idx, 0.0, 1.0)
                z_ref[b:b + 1, l * D:(l + 1) * D] = row * m
        z1 = jnp.dot(z_ref[...], wl1_ref[...], preferred_element_type=jnp.float32)
        z1 = jnp.maximum(z1 + bl1_ref[...], 0.0)
        z2_ref[:, :D] = z1
        z2_ref[:, D:] = cond_ref[...]
        o_ref[...] = jnp.dot(z2_ref[...], wl2_ref[...],
                             preferred_element_type=jnp.float32) + bl2_ref[...]

    return pl.pallas_call(
        body,
        in_specs=[
            pl.BlockSpec(memory_space=pltpu.SMEM),
            pl.BlockSpec((N_PAD, D), lambda: (0, 0)),
            pl.BlockSpec((B, N_PAD), lambda: (0, 0)),
            pl.BlockSpec((B, D_COND), lambda: (0, 0)),
            pl.BlockSpec((D * (1 + L), D), lambda: (0, 0)),
            pl.BlockSpec((1, D), lambda: (0, 0)),
            pl.BlockSpec((D + D_COND, N_C), lambda: (0, 0)),
            pl.BlockSpec((1, N_C), lambda: (0, 0)),
        ],
        out_specs=pl.BlockSpec((B, N_C), lambda: (0, 0)),
        out_shape=jax.ShapeDtypeStruct((B, N_C), jnp.float32),
        scratch_shapes=[
            pltpu.VMEM((B, D * (1 + L)), jnp.float32),
            pltpu.VMEM((B, D + D_COND), jnp.float32),
        ],
    )(base, h5, batch_p, cond, Wl1, bl1.reshape(1, D), Wl2, bl2.reshape(1, N_C))


def kernel(x, edge_index, batch, base, cond, W1, b1, W2, b2, W3, b3, W4, b4,
           W5, b5, Wl1, bl1, Wl2, bl2):
    x_p = jnp.zeros((N_PAD, D), jnp.float32).at[:N].set(x)
    batch_p = jnp.concatenate(
        [batch.astype(jnp.int32), jnp.full((N_PAD - N,), B, jnp.int32)]
    ).reshape(1, N_PAD).astype(jnp.int32)
    batch_b = jnp.broadcast_to(batch_p, (B, N_PAD))

    n_pad_e = E_PAD - E
    pad_idx = (N + (jnp.arange(n_pad_e, dtype=jnp.int32) % (N_PAD - N)))
    src_p = jnp.concatenate([edge_index[0].astype(jnp.int32), pad_idx]
                            ).reshape(NW, NCH, CHUNK)
    dst_p = jnp.concatenate([edge_index[1].astype(jnp.int32), pad_idx]
                            ).reshape(NW, NCH, CHUNK)

    e1 = jnp.zeros((CHUNK, D), jnp.float32).at[:, 0].set(1.0)
    zerosD = jnp.zeros((ROWS_PER_TILE, D), jnp.float32)

    deg_parts = _sc_degree_fn()(dst_p, e1, zerosD)
    g, dinv = _tc_first(x_p, W1, deg_parts)

    sc_agg = _sc_agg_fn()
    for b_prev, W_next in ((b1, W2), (b2, W3), (b3, W4), (b4, W5)):
        acc = sc_agg(g, src_p, dst_p, zerosD)
        g = _tc_mid(acc, g, dinv, b_prev, W_next)
    acc = sc_agg(g, src_p, dst_p, zerosD)
    h5 = _tc_last(acc, g, dinv, b5)

    return _tc_head(h5, batch_b, base.astype(jnp.int32), cond, Wl1, bl1, Wl2, bl2)


# speedup vs baseline: 19.5846x; 1.2765x over previous
import functools

import jax
import jax.numpy as jnp
from jax import lax
from jax.experimental import pallas as pl
from jax.experimental.pallas import tpu as pltpu
from jax.experimental.pallas import tpu_sc as plsc

N = 10000
E = 320000
D = 128
B = 8
L = 16
D_COND = 16
N_C = 8
NPG = N // B

N_PAD = 10240
NW = 32
CHUNK = 128
NCH = (E + NW * CHUNK - 1) // (NW * CHUNK)
E_PAD = NW * NCH * CHUNK
ROWS_PER_TILE = N_PAD // 16
BLK = 1024


@functools.cache
def _sc_mesh():
    return plsc.VectorSubcoreMesh(core_axis_name="c", subcore_axis_name="s",
                                  num_cores=2, num_subcores=16)


@functools.cache
def _sc_degree_fn():
    @functools.partial(
        pl.kernel,
        out_type=jax.ShapeDtypeStruct((2, N_PAD, D), jnp.float32),
        mesh=_sc_mesh(),
        scratch_types=[
            pltpu.VMEM((NCH, CHUNK), jnp.int32),
            pltpu.VMEM((CHUNK, D), jnp.float32),
            pltpu.VMEM_SHARED((N_PAD, D), jnp.float32),
        ],
    )
    def _sc_degree(dst_hbm, e1_hbm, zeros_hbm, out_hbm, dst_v, e1_v, deg_sh):
        c = lax.axis_index("c")
        s = lax.axis_index("s")
        wid = c * 16 + s
        pltpu.sync_copy(dst_hbm.at[wid], dst_v)
        pltpu.sync_copy(e1_hbm, e1_v)
        pltpu.sync_copy(zeros_hbm, deg_sh.at[pl.ds(s * ROWS_PER_TILE, ROWS_PER_TILE)])
        plsc.subcore_barrier()

        def body(j, carry):
            pltpu.sync_copy(e1_v, deg_sh.at[dst_v.at[j]], add=True)
            return carry

        lax.fori_loop(0, NCH, body, 0)
        plsc.subcore_barrier()
        sl = pl.ds(s * ROWS_PER_TILE, ROWS_PER_TILE)
        pltpu.sync_copy(deg_sh.at[sl], out_hbm.at[c].at[sl])

    return _sc_degree


@functools.cache
def _sc_agg_fn():
    @functools.partial(
        pl.kernel,
        out_type=jax.ShapeDtypeStruct((2, N_PAD, D), jnp.float32),
        mesh=_sc_mesh(),
        scratch_types=[
            pltpu.VMEM((NCH, CHUNK), jnp.int32),
            pltpu.VMEM((NCH, CHUNK), jnp.int32),
            pltpu.VMEM((2, CHUNK // 2, D), jnp.float32),
            pltpu.VMEM_SHARED((N_PAD, D), jnp.float32),
            pltpu.SemaphoreType.DMA,
            pltpu.SemaphoreType.DMA,
        ],
    )
    def _sc_agg(g_hbm, src_hbm, dst_hbm, zeros_hbm, out_hbm,
                src_v, dst_v, rows_v, acc_sh, gsem, ssem):
        c = lax.axis_index("c")
        s = lax.axis_index("s")
        wid = c * 16 + s
        CH2 = CHUNK // 2
        NCH2 = NCH * 2
        pltpu.sync_copy(src_hbm.at[wid], src_v)
        pltpu.sync_copy(dst_hbm.at[wid], dst_v)
        pltpu.sync_copy(zeros_hbm, acc_sh.at[pl.ds(s * ROWS_PER_TILE, ROWS_PER_TILE)])
        plsc.subcore_barrier()

        def sidx(v, t):
            return v.at[t // 2, pl.ds((t % 2) * CH2, CH2)]

        pltpu.async_copy(g_hbm.at[sidx(src_v, 0)], rows_v.at[0], gsem)
        pltpu.async_copy(g_hbm.at[sidx(src_v, 1)], rows_v.at[1], gsem)

        def body(t, carry):
            cur = t % 2
            pltpu.make_async_copy(g_hbm.at[sidx(src_v, t)], rows_v.at[cur],
                                  gsem).wait()
            pltpu.async_copy(rows_v.at[cur], acc_sh.at[sidx(dst_v, t)],
                             ssem, add=True)

            @pl.when(t + 2 < NCH2)
            def _():
                pltpu.make_async_copy(
                    rows_v.at[cur], acc_sh.at[sidx(dst_v, t)], ssem).wait()
                pltpu.async_copy(g_hbm.at[sidx(src_v, t + 2)], rows_v.at[cur], gsem)

            return carry

        lax.fori_loop(0, NCH2, body, 0)
        pltpu.make_async_copy(rows_v.at[0], acc_sh.at[sidx(dst_v, 0)], ssem).wait()
        pltpu.make_async_copy(rows_v.at[1], acc_sh.at[sidx(dst_v, 1)], ssem).wait()
        plsc.subcore_barrier()
        sl = pl.ds(s * ROWS_PER_TILE, ROWS_PER_TILE)
        pltpu.sync_copy(acc_sh.at[sl], out_hbm.at[c].at[sl])

    return _sc_agg


def _tc_first(x_p, W1, deg_parts):
    def body(x_ref, w_ref, deg_ref, g_ref, dinv_ref):
        d = deg_ref[0, :, :1] + deg_ref[1, :, :1]
        dinv = lax.rsqrt(d + 1.0)
        t = jnp.dot(x_ref[...], w_ref[...], preferred_element_type=jnp.float32)
        g_ref[...] = t * dinv
        dinv_ref[...] = dinv

    grid = N_PAD // BLK
    return pl.pallas_call(
        body,
        grid=(grid,),
        in_specs=[
            pl.BlockSpec((BLK, D), lambda i: (i, 0)),
            pl.BlockSpec((D, D), lambda i: (0, 0)),
            pl.BlockSpec((2, BLK, D), lambda i: (0, i, 0)),
        ],
        out_specs=[
            pl.BlockSpec((BLK, D), lambda i: (i, 0)),
            pl.BlockSpec((BLK, 1), lambda i: (i, 0)),
        ],
        out_shape=[
            jax.ShapeDtypeStruct((N_PAD, D), jnp.float32),
            jax.ShapeDtypeStruct((N_PAD, 1), jnp.float32),
        ],
    )(x_p, W1, deg_parts)


def _tc_mid(acc, g_in, dinv, b_prev, W_next):
    def body(acc_ref, g_ref, dinv_ref, b_ref, w_ref, o_ref):
        h = (acc_ref[0] + acc_ref[1] + g_ref[...]) * dinv_ref[...] + b_ref[...]
        a = jnp.maximum(h, 0.0)
        t = jnp.dot(a, w_ref[...], preferred_element_type=jnp.float32)
        o_ref[...] = t * dinv_ref[...]

    grid = N_PAD // BLK
    return pl.pallas_call(
        body,
        grid=(grid,),
        in_specs=[
            pl.BlockSpec((2, BLK, D), lambda i: (0, i, 0)),
            pl.BlockSpec((BLK, D), lambda i: (i, 0)),
            pl.BlockSpec((BLK, 1), lambda i: (i, 0)),
            pl.BlockSpec((1, D), lambda i: (0, 0)),
            pl.BlockSpec((D, D), lambda i: (0, 0)),
        ],
        out_specs=pl.BlockSpec((BLK, D), lambda i: (i, 0)),
        out_shape=jax.ShapeDtypeStruct((N_PAD, D), jnp.float32),
    )(acc, g_in, dinv, b_prev.reshape(1, D), W_next)


def _tc_last(acc, g_in, dinv, b5):
    def body(acc_ref, g_ref, dinv_ref, b_ref, o_ref):
        o_ref[...] = (acc_ref[0] + acc_ref[1] + g_ref[...]) * dinv_ref[...] + b_ref[...]

    grid = N_PAD // BLK
    return pl.pallas_call(
        body,
        grid=(grid,),
        in_specs=[
            pl.BlockSpec((2, BLK, D), lambda i: (0, i, 0)),
            pl.BlockSpec((BLK, D), lambda i: (i, 0)),
            pl.BlockSpec((BLK, 1), lambda i: (i, 0)),
            pl.BlockSpec((1, D), lambda i: (0, 0)),
        ],
        out_specs=pl.BlockSpec((BLK, D), lambda i: (i, 0)),
        out_shape=jax.ShapeDtypeStruct((N_PAD, D), jnp.float32),
    )(acc, g_in, dinv, b5.reshape(1, D))


def _tc_head(h5, batch_p, base, cond, Wl1, bl1, Wl2, bl2):
    def body(base_ref, h_ref, batch_ref, cond_ref, wl1_ref, bl1_ref,
             wl2_ref, bl2_ref, o_ref, z_ref, z2_ref):
        row_id = lax.broadcasted_iota(jnp.int32, (B, N_PAD), 0)
        sel = jnp.where(batch_ref[...] == row_id, 1.0, 0.0)
        cnts = jnp.sum(sel, axis=1, keepdims=True)
        xg = jnp.dot(sel, h_ref[...], preferred_element_type=jnp.float32) / cnts
        z_ref[:, L * D:] = xg
        for b in range(B):
            for l in range(L):
                idx = base_ref[b, l]
                row = h_ref[pl.ds(b * NPG + idx, 1), :]
                m = jnp.where(idx != 0, 1.0, 0.0)
                for j in range(l):
                    m = m * jnp.where(base_ref[b, j] == idx, 0.0, 1.0)
                z_ref[b:b + 1, l * D:(l + 1) * D] = row * m
        z1 = jnp.dot(z_ref[...], wl1_ref[...], preferred_element_type=jnp.float32)
        z1 = jnp.maximum(z1 + bl1_ref[...], 0.0)
        z2_ref[:, :D] = z1
        z2_ref[:, D:] = cond_ref[...]
        o_ref[...] = jnp.dot(z2_ref[...], wl2_ref[...],
                             preferred_element_type=jnp.float32) + bl2_ref[...]

    return pl.pallas_call(
        body,
        in_specs=[
            pl.BlockSpec(memory_space=pltpu.SMEM),
            pl.BlockSpec((N_PAD, D), lambda: (0, 0)),
            pl.BlockSpec((B, N_PAD), lambda: (0, 0)),
            pl.BlockSpec((B, D_COND), lambda: (0, 0)),
            pl.BlockSpec((D * (1 + L), D), lambda: (0, 0)),
            pl.BlockSpec((1, D), lambda: (0, 0)),
            pl.BlockSpec((D + D_COND, N_C), lambda: (0, 0)),
            pl.BlockSpec((1, N_C), lambda: (0, 0)),
        ],
        out_specs=pl.BlockSpec((B, N_C), lambda: (0, 0)),
        out_shape=jax.ShapeDtypeStruct((B, N_C), jnp.float32),
        scratch_shapes=[
            pltpu.VMEM((B, D * (1 + L)), jnp.float32),
            pltpu.VMEM((B, D + D_COND), jnp.float32),
        ],
    )(base, h5, batch_p, cond, Wl1, bl1.reshape(1, D), Wl2, bl2.reshape(1, N_C))


def kernel(x, edge_index, batch, base, cond, W1, b1, W2, b2, W3, b3, W4, b4,
           W5, b5, Wl1, bl1, Wl2, bl2):
    x_p = jnp.zeros((N_PAD, D), jnp.float32).at[:N].set(x)
    batch_p = jnp.concatenate(
        [batch.astype(jnp.int32), jnp.full((N_PAD - N,), B, jnp.int32)]
    ).reshape(1, N_PAD).astype(jnp.int32)
    batch_b = jnp.broadcast_to(batch_p, (B, N_PAD))

    n_pad_e = E_PAD - E
    pad_idx = (N + (jnp.arange(n_pad_e, dtype=jnp.int32) % (N_PAD - N)))
    src_p = jnp.concatenate([edge_index[0].astype(jnp.int32), pad_idx]
                            ).reshape(NW, NCH, CHUNK)
    dst_p = jnp.concatenate([edge_index[1].astype(jnp.int32), pad_idx]
                            ).reshape(NW, NCH, CHUNK)

    e1 = jnp.zeros((CHUNK, D), jnp.float32).at[:, 0].set(1.0)
    zerosD = jnp.zeros((ROWS_PER_TILE, D), jnp.float32)

    deg_parts = _sc_degree_fn()(dst_p, e1, zerosD)
    g, dinv = _tc_first(x_p, W1, deg_parts)

    sc_agg = _sc_agg_fn()
    for b_prev, W_next in ((b1, W2), (b2, W3), (b3, W4), (b4, W5)):
        acc = sc_agg(g, src_p, dst_p, zerosD)
        g = _tc_mid(acc, g, dinv, b_prev, W_next)
    acc = sc_agg(g, src_p, dst_p, zerosD)
    h5 = _tc_last(acc, g, dinv, b5)

    return _tc_head(h5, batch_b, base.astype(jnp.int32), cond, Wl1, bl1, Wl2, bl2)


# speedup vs baseline: 22.4778x; 1.1477x over previous
import functools

import jax
import jax.numpy as jnp
from jax import lax
from jax.experimental import pallas as pl
from jax.experimental.pallas import tpu as pltpu
from jax.experimental.pallas import tpu_sc as plsc

N = 10000
E = 320000
D = 128
B = 8
L = 16
D_COND = 16
N_C = 8
NPG = N // B

N_PAD = 10240
NW = 32
CHUNK = 128
NCH = (E + NW * CHUNK - 1) // (NW * CHUNK)
E_PAD = NW * NCH * CHUNK
ROWS_PER_TILE = N_PAD // 16
BLK = 1024


@functools.cache
def _sc_mesh():
    return plsc.VectorSubcoreMesh(core_axis_name="c", subcore_axis_name="s",
                                  num_cores=2, num_subcores=16)


@functools.cache
def _sc_degree_fn():
    @functools.partial(
        pl.kernel,
        out_type=jax.ShapeDtypeStruct((2, N_PAD, D), jnp.float32),
        mesh=_sc_mesh(),
        scratch_types=[
            pltpu.VMEM((NCH, CHUNK), jnp.int32),
            pltpu.VMEM((CHUNK, D), jnp.float32),
            pltpu.VMEM_SHARED((N_PAD, D), jnp.float32),
            pltpu.SemaphoreType.DMA,
        ],
    )
    def _sc_degree(dst_hbm, e1_hbm, zeros_hbm, out_hbm, dst_v, e1_v, deg_sh, ssem):
        c = lax.axis_index("c")
        s = lax.axis_index("s")
        wid = c * 16 + s
        pltpu.sync_copy(dst_hbm.at[wid], dst_v)
        pltpu.sync_copy(e1_hbm, e1_v)
        pltpu.sync_copy(zeros_hbm, deg_sh.at[pl.ds(s * ROWS_PER_TILE, ROWS_PER_TILE)])
        plsc.subcore_barrier()

        def body(j, carry):
            @pl.when(j >= 4)
            def _():
                pltpu.make_async_copy(e1_v, deg_sh.at[dst_v.at[0]], ssem).wait()

            pltpu.async_copy(e1_v, deg_sh.at[dst_v.at[j]], ssem, add=True)
            return carry

        lax.fori_loop(0, NCH, body, 0)
        for _ in range(4):
            pltpu.make_async_copy(e1_v, deg_sh.at[dst_v.at[0]], ssem).wait()
        plsc.subcore_barrier()
        sl = pl.ds(s * ROWS_PER_TILE, ROWS_PER_TILE)
        pltpu.sync_copy(deg_sh.at[sl], out_hbm.at[c].at[sl])

    return _sc_degree


@functools.cache
def _sc_agg_fn():
    @functools.partial(
        pl.kernel,
        out_type=jax.ShapeDtypeStruct((2, N_PAD, D), jnp.float32),
        mesh=_sc_mesh(),
        scratch_types=[
            pltpu.VMEM((NCH, CHUNK), jnp.int32),
            pltpu.VMEM((NCH, CHUNK), jnp.int32),
            pltpu.VMEM((3, CHUNK // 2, D), jnp.float32),
            pltpu.VMEM_SHARED((N_PAD, D), jnp.float32),
            pltpu.SemaphoreType.DMA,
            pltpu.SemaphoreType.DMA,
        ],
    )
    def _sc_agg(g_hbm, src_hbm, dst_hbm, zeros_hbm, out_hbm,
                src_v, dst_v, rows_v, acc_sh, gsem, ssem):
        c = lax.axis_index("c")
        s = lax.axis_index("s")
        wid = c * 16 + s
        CH2 = CHUNK // 2
        NCH2 = NCH * 2
        pltpu.sync_copy(src_hbm.at[wid], src_v)
        pltpu.sync_copy(dst_hbm.at[wid], dst_v)
        pltpu.sync_copy(zeros_hbm, acc_sh.at[pl.ds(s * ROWS_PER_TILE, ROWS_PER_TILE)])
        plsc.subcore_barrier()

        def sidx(v, t):
            return v.at[t // 2, pl.ds((t % 2) * CH2, CH2)]

        pltpu.async_copy(g_hbm.at[sidx(src_v, 0)], rows_v.at[0], gsem)
        pltpu.async_copy(g_hbm.at[sidx(src_v, 1)], rows_v.at[1], gsem)

        def body(t, carry):
            b = t % 3
            pltpu.make_async_copy(g_hbm.at[sidx(src_v, t)], rows_v.at[b],
                                  gsem).wait()
            pltpu.async_copy(rows_v.at[b], acc_sh.at[sidx(dst_v, t)],
                             ssem, add=True)

            @pl.when((t >= 1) & (t + 2 < NCH2))
            def _():
                pltpu.make_async_copy(
                    rows_v.at[(t - 1) % 3], acc_sh.at[sidx(dst_v, t - 1)],
                    ssem).wait()

            @pl.when(t + 2 < NCH2)
            def _():
                pltpu.async_copy(g_hbm.at[sidx(src_v, t + 2)],
                                 rows_v.at[(t + 2) % 3], gsem)

            return carry

        lax.fori_loop(0, NCH2, body, 0)
        for k in (NCH2 - 3, NCH2 - 2, NCH2 - 1):
            pltpu.make_async_copy(rows_v.at[k % 3], acc_sh.at[sidx(dst_v, k)],
                                  ssem).wait()
        plsc.subcore_barrier()
        sl = pl.ds(s * ROWS_PER_TILE, ROWS_PER_TILE)
        pltpu.sync_copy(acc_sh.at[sl], out_hbm.at[c].at[sl])

    return _sc_agg


def _tc_first(x_p, W1, deg_parts):
    def body(x_ref, w_ref, deg_ref, g_ref, dinv_ref):
        d = deg_ref[0, :, :1] + deg_ref[1, :, :1]
        dinv = lax.rsqrt(d + 1.0)
        t = jnp.dot(x_ref[...], w_ref[...], preferred_element_type=jnp.float32)
        g_ref[...] = t * dinv
        dinv_ref[...] = dinv

    grid = N_PAD // BLK
    return pl.pallas_call(
        body,
        grid=(grid,),
        in_specs=[
            pl.BlockSpec((BLK, D), lambda i: (i, 0)),
            pl.BlockSpec((D, D), lambda i: (0, 0)),
            pl.BlockSpec((2, BLK, D), lambda i: (0, i, 0)),
        ],
        out_specs=[
            pl.BlockSpec((BLK, D), lambda i: (i, 0)),
            pl.BlockSpec((BLK, 1), lambda i: (i, 0)),
        ],
        out_shape=[
            jax.ShapeDtypeStruct((N_PAD, D), jnp.float32),
            jax.ShapeDtypeStruct((N_PAD, 1), jnp.float32),
        ],
    )(x_p, W1, deg_parts)


def _tc_mid(acc, g_in, dinv, b_prev, W_next):
    def body(acc_ref, g_ref, dinv_ref, b_ref, w_ref, o_ref):
        h = (acc_ref[0] + acc_ref[1] + g_ref[...]) * dinv_ref[...] + b_ref[...]
        a = jnp.maximum(h, 0.0)
        t = jnp.dot(a, w_ref[...], preferred_element_type=jnp.float32)
        o_ref[...] = t * dinv_ref[...]

    grid = N_PAD // BLK
    return pl.pallas_call(
        body,
        grid=(grid,),
        in_specs=[
            pl.BlockSpec((2, BLK, D), lambda i: (0, i, 0)),
            pl.BlockSpec((BLK, D), lambda i: (i, 0)),
            pl.BlockSpec((BLK, 1), lambda i: (i, 0)),
            pl.BlockSpec((1, D), lambda i: (0, 0)),
            pl.BlockSpec((D, D), lambda i: (0, 0)),
        ],
        out_specs=pl.BlockSpec((BLK, D), lambda i: (i, 0)),
        out_shape=jax.ShapeDtypeStruct((N_PAD, D), jnp.float32),
    )(acc, g_in, dinv, b_prev.reshape(1, D), W_next)


def _tc_last(acc, g_in, dinv, b5):
    def body(acc_ref, g_ref, dinv_ref, b_ref, o_ref):
        o_ref[...] = (acc_ref[0] + acc_ref[1] + g_ref[...]) * dinv_ref[...] + b_ref[...]

    grid = N_PAD // BLK
    return pl.pallas_call(
        body,
        grid=(grid,),
        in_specs=[
            pl.BlockSpec((2, BLK, D), lambda i: (0, i, 0)),
            pl.BlockSpec((BLK, D), lambda i: (i, 0)),
            pl.BlockSpec((BLK, 1), lambda i: (i, 0)),
            pl.BlockSpec((1, D), lambda i: (0, 0)),
        ],
        out_specs=pl.BlockSpec((BLK, D), lambda i: (i, 0)),
        out_shape=jax.ShapeDtypeStruct((N_PAD, D), jnp.float32),
    )(acc, g_in, dinv, b5.reshape(1, D))


def _tc_head(h5, batch_p, base, cond, Wl1, bl1, Wl2, bl2):
    def body(base_ref, h_ref, batch_ref, cond_ref, wl1_ref, bl1_ref,
             wl2_ref, bl2_ref, o_ref, z_ref, z2_ref):
        row_id = lax.broadcasted_iota(jnp.int32, (B, N_PAD), 0)
        sel = jnp.where(batch_ref[...] == row_id, 1.0, 0.0)
        cnts = jnp.sum(sel, axis=1, keepdims=True)
        xg = jnp.dot(sel, h_ref[...], preferred_element_type=jnp.float32) / cnts
        z_ref[:, L * D:] = xg
        for b in range(B):
            for l in range(L):
                idx = base_ref[b, l]
                row = h_ref[pl.ds(b * NPG + idx, 1), :]
                m = jnp.where(idx != 0, 1.0, 0.0)
                for j in range(l):
                    m = m * jnp.where(base_ref[b, j] == idx, 0.0, 1.0)
                z_ref[b:b + 1, l * D:(l + 1) * D] = row * m
        z1 = jnp.dot(z_ref[...], wl1_ref[...], preferred_element_type=jnp.float32)
        z1 = jnp.maximum(z1 + bl1_ref[...], 0.0)
        z2_ref[:, :D] = z1
        z2_ref[:, D:] = cond_ref[...]
        o_ref[...] = jnp.dot(z2_ref[...], wl2_ref[...],
                             preferred_element_type=jnp.float32) + bl2_ref[...]

    return pl.pallas_call(
        body,
        in_specs=[
            pl.BlockSpec(memory_space=pltpu.SMEM),
            pl.BlockSpec((N_PAD, D), lambda: (0, 0)),
            pl.BlockSpec((B, N_PAD), lambda: (0, 0)),
            pl.BlockSpec((B, D_COND), lambda: (0, 0)),
            pl.BlockSpec((D * (1 + L), D), lambda: (0, 0)),
            pl.BlockSpec((1, D), lambda: (0, 0)),
            pl.BlockSpec((D + D_COND, N_C), lambda: (0, 0)),
            pl.BlockSpec((1, N_C), lambda: (0, 0)),
        ],
        out_specs=pl.BlockSpec((B, N_C), lambda: (0, 0)),
        out_shape=jax.ShapeDtypeStruct((B, N_C), jnp.float32),
        scratch_shapes=[
            pltpu.VMEM((B, D * (1 + L)), jnp.float32),
            pltpu.VMEM((B, D + D_COND), jnp.float32),
        ],
    )(base, h5, batch_p, cond, Wl1, bl1.reshape(1, D), Wl2, bl2.reshape(1, N_C))


def kernel(x, edge_index, batch, base, cond, W1, b1, W2, b2, W3, b3, W4, b4,
           W5, b5, Wl1, bl1, Wl2, bl2):
    x_p = jnp.zeros((N_PAD, D), jnp.float32).at[:N].set(x)
    batch_p = jnp.concatenate(
        [batch.astype(jnp.int32), jnp.full((N_PAD - N,), B, jnp.int32)]
    ).reshape(1, N_PAD).astype(jnp.int32)
    batch_b = jnp.broadcast_to(batch_p, (B, N_PAD))

    n_pad_e = E_PAD - E
    pad_idx = (N + (jnp.arange(n_pad_e, dtype=jnp.int32) % (N_PAD - N)))
    src_p = jnp.concatenate([edge_index[0].astype(jnp.int32), pad_idx]
                            ).reshape(NW, NCH, CHUNK)
    dst_p = jnp.concatenate([edge_index[1].astype(jnp.int32), pad_idx]
                            ).reshape(NW, NCH, CHUNK)

    e1 = jnp.zeros((CHUNK, D), jnp.float32).at[:, 0].set(1.0)
    zerosD = jnp.zeros((ROWS_PER_TILE, D), jnp.float32)

    deg_parts = _sc_degree_fn()(dst_p, e1, zerosD)
    g, dinv = _tc_first(x_p, W1, deg_parts)

    sc_agg = _sc_agg_fn()
    for b_prev, W_next in ((b1, W2), (b2, W3), (b3, W4), (b4, W5)):
        acc = sc_agg(g, src_p, dst_p, zerosD)
        g = _tc_mid(acc, g, dinv, b_prev, W_next)
    acc = sc_agg(g, src_p, dst_p, zerosD)
    h5 = _tc_last(acc, g, dinv, b5)

    return _tc_head(h5, batch_b, base.astype(jnp.int32), cond, Wl1, bl1, Wl2, bl2)


# speedup vs baseline: 22.9038x; 1.0190x over previous
import functools

import jax
import jax.numpy as jnp
from jax import lax
from jax.experimental import pallas as pl
from jax.experimental.pallas import tpu as pltpu
from jax.experimental.pallas import tpu_sc as plsc

N = 10000
E = 320000
D = 128
B = 8
L = 16
D_COND = 16
N_C = 8
NPG = N // B

N_PAD = 10240
NW = 32
CHUNK = 128
NCH = (E + NW * CHUNK - 1) // (NW * CHUNK)
E_PAD = NW * NCH * CHUNK
ROWS_PER_TILE = N_PAD // 16
BLK = 1024


@functools.cache
def _sc_mesh():
    return plsc.VectorSubcoreMesh(core_axis_name="c", subcore_axis_name="s",
                                  num_cores=2, num_subcores=16)


@functools.cache
def _sc_degree_fn():
    @functools.partial(
        pl.kernel,
        out_type=jax.ShapeDtypeStruct((2, N_PAD, D), jnp.float32),
        mesh=_sc_mesh(),
        scratch_types=[
            pltpu.VMEM((NCH, CHUNK), jnp.int32),
            pltpu.VMEM((CHUNK, D), jnp.float32),
            pltpu.VMEM_SHARED((N_PAD, D), jnp.float32),
            pltpu.SemaphoreType.DMA,
        ],
    )
    def _sc_degree(dst_hbm, e1_hbm, zeros_hbm, out_hbm, dst_v, e1_v, deg_sh, ssem):
        c = lax.axis_index("c")
        s = lax.axis_index("s")
        wid = c * 16 + s
        zsl = pl.ds(s * ROWS_PER_TILE, ROWS_PER_TILE)
        pltpu.async_copy(dst_hbm.at[wid], dst_v, ssem)
        pltpu.async_copy(e1_hbm, e1_v, ssem)
        pltpu.async_copy(zeros_hbm, deg_sh.at[zsl], ssem)
        pltpu.make_async_copy(dst_hbm.at[wid], dst_v, ssem).wait()
        pltpu.make_async_copy(e1_hbm, e1_v, ssem).wait()
        pltpu.make_async_copy(zeros_hbm, deg_sh.at[zsl], ssem).wait()
        plsc.subcore_barrier()

        def body(j, carry):
            @pl.when(j >= 4)
            def _():
                pltpu.make_async_copy(e1_v, deg_sh.at[dst_v.at[0]], ssem).wait()

            pltpu.async_copy(e1_v, deg_sh.at[dst_v.at[j]], ssem, add=True)
            return carry

        lax.fori_loop(0, NCH, body, 0)
        for _ in range(4):
            pltpu.make_async_copy(e1_v, deg_sh.at[dst_v.at[0]], ssem).wait()
        plsc.subcore_barrier()
        sl = pl.ds(s * ROWS_PER_TILE, ROWS_PER_TILE)
        pltpu.sync_copy(deg_sh.at[sl], out_hbm.at[c].at[sl])

    return _sc_degree


@functools.cache
def _sc_agg_fn():
    @functools.partial(
        pl.kernel,
        out_type=jax.ShapeDtypeStruct((2, N_PAD, D), jnp.float32),
        mesh=_sc_mesh(),
        scratch_types=[
            pltpu.VMEM((NCH, CHUNK), jnp.int32),
            pltpu.VMEM((NCH, CHUNK), jnp.int32),
            pltpu.VMEM((3, CHUNK // 2, D), jnp.float32),
            pltpu.VMEM_SHARED((N_PAD, D), jnp.float32),
            pltpu.SemaphoreType.DMA,
            pltpu.SemaphoreType.DMA,
        ],
    )
    def _sc_agg(g_hbm, src_hbm, dst_hbm, zeros_hbm, out_hbm,
                src_v, dst_v, rows_v, acc_sh, gsem, ssem):
        c = lax.axis_index("c")
        s = lax.axis_index("s")
        wid = c * 16 + s
        CH2 = CHUNK // 2
        NCH2 = NCH * 2
        zsl = pl.ds(s * ROWS_PER_TILE, ROWS_PER_TILE)
        pltpu.async_copy(src_hbm.at[wid], src_v, gsem)
        pltpu.async_copy(dst_hbm.at[wid], dst_v, gsem)
        pltpu.async_copy(zeros_hbm, acc_sh.at[zsl], gsem)
        pltpu.make_async_copy(src_hbm.at[wid], src_v, gsem).wait()
        pltpu.make_async_copy(dst_hbm.at[wid], dst_v, gsem).wait()
        pltpu.make_async_copy(zeros_hbm, acc_sh.at[zsl], gsem).wait()
        plsc.subcore_barrier()

        def sidx(v, t):
            return v.at[t // 2, pl.ds((t % 2) * CH2, CH2)]

        pltpu.async_copy(g_hbm.at[sidx(src_v, 0)], rows_v.at[0], gsem)
        pltpu.async_copy(g_hbm.at[sidx(src_v, 1)], rows_v.at[1], gsem)

        def body(t, carry):
            b = t % 3
            pltpu.make_async_copy(g_hbm.at[sidx(src_v, t)], rows_v.at[b],
                                  gsem).wait()
            pltpu.async_copy(rows_v.at[b], acc_sh.at[sidx(dst_v, t)],
                             ssem, add=True)

            @pl.when((t >= 1) & (t + 2 < NCH2))
            def _():
                pltpu.make_async_copy(
                    rows_v.at[(t - 1) % 3], acc_sh.at[sidx(dst_v, t - 1)],
                    ssem).wait()

            @pl.when(t + 2 < NCH2)
            def _():
                pltpu.async_copy(g_hbm.at[sidx(src_v, t + 2)],
                                 rows_v.at[(t + 2) % 3], gsem)

            return carry

        lax.fori_loop(0, NCH2, body, 0)
        for k in (NCH2 - 3, NCH2 - 2, NCH2 - 1):
            pltpu.make_async_copy(rows_v.at[k % 3], acc_sh.at[sidx(dst_v, k)],
                                  ssem).wait()
        plsc.subcore_barrier()
        sl = pl.ds(s * ROWS_PER_TILE, ROWS_PER_TILE)
        pltpu.sync_copy(acc_sh.at[sl], out_hbm.at[c].at[sl])

    return _sc_agg


def _tc_matmul(x_p, W1):
    def body(x_ref, w_ref, o_ref):
        o_ref[...] = jnp.dot(x_ref[...], w_ref[...],
                             preferred_element_type=jnp.float32)

    grid = N_PAD // BLK
    return pl.pallas_call(
        body,
        grid=(grid,),
        in_specs=[
            pl.BlockSpec((BLK, D), lambda i: (i, 0)),
            pl.BlockSpec((D, D), lambda i: (0, 0)),
        ],
        out_specs=pl.BlockSpec((BLK, D), lambda i: (i, 0)),
        out_shape=jax.ShapeDtypeStruct((N_PAD, D), jnp.float32),
    )(x_p, W1)


def _tc_scale(t1, deg_parts):
    def body(t_ref, deg_ref, g_ref, dinv_ref):
        d = deg_ref[0, :, :1] + deg_ref[1, :, :1]
        dinv = lax.rsqrt(d + 1.0)
        g_ref[...] = t_ref[...] * dinv
        dinv_ref[...] = dinv

    grid = N_PAD // BLK
    return pl.pallas_call(
        body,
        grid=(grid,),
        in_specs=[
            pl.BlockSpec((BLK, D), lambda i: (i, 0)),
            pl.BlockSpec((2, BLK, D), lambda i: (0, i, 0)),
        ],
        out_specs=[
            pl.BlockSpec((BLK, D), lambda i: (i, 0)),
            pl.BlockSpec((BLK, 1), lambda i: (i, 0)),
        ],
        out_shape=[
            jax.ShapeDtypeStruct((N_PAD, D), jnp.float32),
            jax.ShapeDtypeStruct((N_PAD, 1), jnp.float32),
        ],
    )(t1, deg_parts)


def _tc_mid(acc, g_in, dinv, b_prev, W_next):
    def body(acc_ref, g_ref, dinv_ref, b_ref, w_ref, o_ref):
        h = (acc_ref[0] + acc_ref[1] + g_ref[...]) * dinv_ref[...] + b_ref[...]
        a = jnp.maximum(h, 0.0)
        t = jnp.dot(a, w_ref[...], preferred_element_type=jnp.float32)
        o_ref[...] = t * dinv_ref[...]

    grid = N_PAD // BLK
    return pl.pallas_call(
        body,
        grid=(grid,),
        in_specs=[
            pl.BlockSpec((2, BLK, D), lambda i: (0, i, 0)),
            pl.BlockSpec((BLK, D), lambda i: (i, 0)),
            pl.BlockSpec((BLK, 1), lambda i: (i, 0)),
            pl.BlockSpec((1, D), lambda i: (0, 0)),
            pl.BlockSpec((D, D), lambda i: (0, 0)),
        ],
        out_specs=pl.BlockSpec((BLK, D), lambda i: (i, 0)),
        out_shape=jax.ShapeDtypeStruct((N_PAD, D), jnp.float32),
    )(acc, g_in, dinv, b_prev.reshape(1, D), W_next)


def _tc_head(acc, g_in, dinv, b5, batch_p, base, cond, Wl1, bl1, Wl2, bl2):
    def body(base_ref, acc_ref, g_ref, dinv_ref, b5_ref, batch_ref, cond_ref,
             wl1_ref, bl1_ref, wl2_ref, bl2_ref, o_ref, h_ref, z_ref, z2_ref):
        h_ref[...] = ((acc_ref[0] + acc_ref[1] + g_ref[...]) * dinv_ref[...]
                      + b5_ref[...])
        row_id = lax.broadcasted_iota(jnp.int32, (B, N_PAD), 0)
        sel = jnp.where(batch_ref[...] == row_id, 1.0, 0.0)
        cnts = jnp.sum(sel, axis=1, keepdims=True)
        xg = jnp.dot(sel, h_ref[...], preferred_element_type=jnp.float32) / cnts
        z_ref[:, L * D:] = xg
        for b in range(B):
            for l in range(L):
                idx = base_ref[b, l]
                row = h_ref[pl.ds(b * NPG + idx, 1), :]
                m = jnp.where(idx != 0, 1.0, 0.0)
                for j in range(l):
                    m = m * jnp.where(base_ref[b, j] == idx, 0.0, 1.0)
                z_ref[b:b + 1, l * D:(l + 1) * D] = row * m
        z1 = jnp.dot(z_ref[...], wl1_ref[...], preferred_element_type=jnp.float32)
        z1 = jnp.maximum(z1 + bl1_ref[...], 0.0)
        z2_ref[:, :D] = z1
        z2_ref[:, D:] = cond_ref[...]
        o_ref[...] = jnp.dot(z2_ref[...], wl2_ref[...],
                             preferred_element_type=jnp.float32) + bl2_ref[...]

    return pl.pallas_call(
        body,
        in_specs=[
            pl.BlockSpec(memory_space=pltpu.SMEM),
            pl.BlockSpec((2, N_PAD, D), lambda: (0, 0, 0)),
            pl.BlockSpec((N_PAD, D), lambda: (0, 0)),
            pl.BlockSpec((N_PAD, 1), lambda: (0, 0)),
            pl.BlockSpec((1, D), lambda: (0, 0)),
            pl.BlockSpec((B, N_PAD), lambda: (0, 0)),
            pl.BlockSpec((B, D_COND), lambda: (0, 0)),
            pl.BlockSpec((D * (1 + L), D), lambda: (0, 0)),
            pl.BlockSpec((1, D), lambda: (0, 0)),
            pl.BlockSpec((D + D_COND, N_C), lambda: (0, 0)),
            pl.BlockSpec((1, N_C), lambda: (0, 0)),
        ],
        out_specs=pl.BlockSpec((B, N_C), lambda: (0, 0)),
        out_shape=jax.ShapeDtypeStruct((B, N_C), jnp.float32),
        scratch_shapes=[
            pltpu.VMEM((N_PAD, D), jnp.float32),
            pltpu.VMEM((B, D * (1 + L)), jnp.float32),
            pltpu.VMEM((B, D + D_COND), jnp.float32),
        ],
    )(base, acc, g_in, dinv, b5.reshape(1, D), batch_p, cond,
      Wl1, bl1.reshape(1, D), Wl2, bl2.reshape(1, N_C))


def kernel(x, edge_index, batch, base, cond, W1, b1, W2, b2, W3, b3, W4, b4,
           W5, b5, Wl1, bl1, Wl2, bl2):
    x_p = jnp.zeros((N_PAD, D), jnp.float32).at[:N].set(x)
    batch_p = jnp.concatenate(
        [batch.astype(jnp.int32), jnp.full((N_PAD - N,), B, jnp.int32)]
    ).reshape(1, N_PAD).astype(jnp.int32)
    batch_b = jnp.broadcast_to(batch_p, (B, N_PAD))

    n_pad_e = E_PAD - E
    pad_idx = (N + (jnp.arange(n_pad_e, dtype=jnp.int32) % (N_PAD - N)))
    src_p = jnp.concatenate([edge_index[0].astype(jnp.int32), pad_idx]
                            ).reshape(NW, NCH, CHUNK)
    dst_p = jnp.concatenate([edge_index[1].astype(jnp.int32), pad_idx]
                            ).reshape(NW, NCH, CHUNK)

    e1 = jnp.zeros((CHUNK, D), jnp.float32).at[:, 0].set(1.0)
    zerosD = jnp.zeros((ROWS_PER_TILE, D), jnp.float32)

    t1 = _tc_matmul(x_p, W1)
    deg_parts = _sc_degree_fn()(dst_p, e1, zerosD)
    g, dinv = _tc_scale(t1, deg_parts)

    sc_agg = _sc_agg_fn()
    for b_prev, W_next in ((b1, W2), (b2, W3), (b3, W4), (b4, W5)):
        acc = sc_agg(g, src_p, dst_p, zerosD)
        g = _tc_mid(acc, g, dinv, b_prev, W_next)
    acc = sc_agg(g, src_p, dst_p, zerosD)

    return _tc_head(acc, g, dinv, b5, batch_b, base.astype(jnp.int32), cond,
                    Wl1, bl1, Wl2, bl2)
